# baseline (speedup 1.0000x reference)
import functools

import jax
import jax.numpy as jnp
from jax import lax
from jax.experimental import pallas as pl
from jax.experimental.pallas import tpu as pltpu
from jax.experimental.pallas import tpu_sc as plsc

NC = 2
NS = 16
NW = NC * NS
L = 16
D = 64
DW = D // 4
BT = 128


def _sc_dequant_gather(flat_idx, table_w, scales, batch, hist):
  bt_per_w = batch // BT // NW
  ET = D // 8

  mesh = plsc.VectorSubcoreMesh(
      core_axis_name="c", subcore_axis_name="s", num_cores=NC, num_subcores=NS
  )

  @functools.partial(
      pl.kernel,
      out_type=jax.ShapeDtypeStruct((hist, ET, batch // BT, 8, BT),
                                    jnp.float32),
      mesh=mesh,
      scratch_types=[
          pltpu.VMEM((BT * hist,), jnp.int32),
          pltpu.VMEM((BT,), jnp.int32),
          pltpu.VMEM((BT, DW), jnp.int32),
          pltpu.VMEM((BT,), jnp.float32),
          pltpu.VMEM((ET, 8, BT), jnp.float32),
          pltpu.SemaphoreType.DMA,
          pltpu.SemaphoreType.DMA,
          pltpu.SemaphoreType.DMA,
      ],
      compiler_params=pltpu.CompilerParams(
          use_tc_tiling_on_sc=False, needs_layout_passes=False),
  )
  def body(idx_hbm, tab_hbm, scl_hbm, out_hbm, idxb_v, idxh_v, rows_v, scl_v,
           tile_v, sem_r, sem_s, sem_o):
    wid = lax.axis_index("s") * NC + lax.axis_index("c")
    lanes = jnp.arange(L, dtype=jnp.int32)
    col_hi = [(lanes * 4 + k) // 8 for k in range(4)]
    col_lo = [(lanes * 4 + k) % 8 for k in range(4)]

    def blk_body(blk, carry):
      bt = wid * bt_per_w + blk
      off = bt * BT * hist
      pltpu.sync_copy(idx_hbm.at[pl.ds(off, BT * hist)], idxb_v)

      def h_body(h, carry2):
        for j in range(BT // L):
          g = plsc.load_gather(idxb_v, [(j * L + lanes) * hist + h])
          idxh_v[pl.ds(j * L, L)] = g
        cp_r = pltpu.async_copy(tab_hbm.at[idxh_v], rows_v, sem_r)
        cp_s = pltpu.async_copy(scl_hbm.at[idxh_v], scl_v, sem_s)
        cp_r.wait()
        cp_s.wait()

        def row_body(r, carry3):
          rsplat = jnp.full((L,), r, dtype=jnp.int32)
          w = rows_v[r, :]
          s = plsc.load_gather(scl_v, [rsplat])
          for k in range(4):
            if k < 3:
              v = (w << (24 - 8 * k)) >> 24
            else:
              v = w >> 24
            plsc.store_scatter(tile_v, [col_hi[k], col_lo[k], rsplat],
                               v.astype(jnp.float32) * s)
          return carry3

        lax.fori_loop(0, BT, row_body, 0)
        cps = [pltpu.async_copy(tile_v.at[et], out_hbm.at[h, et, bt], sem_o)
               for et in range(ET)]
        for cp in cps:
          cp.wait()
        return carry2

      lax.fori_loop(0, hist, h_body, 0)
      return carry

    lax.fori_loop(0, bt_per_w, blk_body, 0)

  return body(flat_idx, table_w, scales)


def kernel(indices, q_table, scales):
  batch, hist = indices.shape
  flat_idx = indices.reshape(-1).astype(jnp.int32)
  w16 = lax.bitcast_convert_type(q_table.reshape(-1, DW, 4), jnp.int32)
  w128 = lax.optimization_barrier(w16.reshape(-1, 128))
  table_w = w128.reshape(-1, DW)
  out5 = _sc_dequant_gather(flat_idx, table_w, scales, batch, hist)
  return out5.transpose(2, 4, 0, 1, 3).reshape(batch, hist, D)
